# baseline (speedup 1.0000x reference)
import functools

import jax
import jax.numpy as jnp
from jax import lax
from jax.experimental import pallas as pl
from jax.experimental.pallas import tpu as pltpu
from jax.experimental.pallas import tpu_sc as plsc

B, S, V, D = 4096, 50, 1000000, 32
N = B * S

_info = plsc.get_sparse_core_info()
NC, NS = _info.num_cores, _info.num_subcores
NW = NC * NS

SRC_PER_W = N // NW
TGT_PER_W = 3 * N // NW
CH = 1600


def _sc_gather():
  mesh = plsc.VectorSubcoreMesh(core_axis_name="c", subcore_axis_name="s")

  @functools.partial(
      pl.kernel,
      mesh=mesh,
      compiler_params=pltpu.CompilerParams(use_tc_tiling_on_sc=False),
      out_type=[
          jax.ShapeDtypeStruct((N, D), jnp.float32),
          jax.ShapeDtypeStruct((3 * N, D), jnp.float32),
      ],
      scratch_types=[
          pltpu.VMEM((CH,), jnp.int32),
          pltpu.VMEM((CH, D), jnp.float32),
          pltpu.SemaphoreType.DMA,
      ],
  )
  def k(w_src, w_tgt, src_idx, tgt_idx, out_src, out_tgt, idx_v, rows_v, sem):
    wid = lax.axis_index("s") * NC + lax.axis_index("c")

    def run(table, idx_hbm, out_hbm, per_w):
      for c in range(per_w // CH):
        base = wid * per_w + c * CH
        pltpu.sync_copy(idx_hbm.at[pl.ds(base, CH)], idx_v)
        pltpu.async_copy(table.at[idx_v], rows_v, sem).wait()
        pltpu.sync_copy(rows_v, out_hbm.at[pl.ds(base, CH)])

    run(w_src, src_idx, out_src, SRC_PER_W)
    run(w_tgt, tgt_idx, out_tgt, TGT_PER_W)

  return k


def _mask_body(m_ref, o_ref):
  o_ref[...] = m_ref[...] == 0


def kernel(sources_input_ids, sources_attention_mask,
           hypotheses_input_ids, hypotheses_attention_mask,
           ref0_input_ids, ref0_attention_mask,
           ref1_input_ids, ref1_attention_mask,
           W_src, W_tgt):
  src_idx = sources_input_ids.reshape(N).astype(jnp.int32)
  tgt_idx = jnp.concatenate([
      hypotheses_input_ids.reshape(N),
      ref0_input_ids.reshape(N),
      ref1_input_ids.reshape(N),
  ]).astype(jnp.int32)

  emb_src, emb_tgt = _sc_gather()(W_src, W_tgt, src_idx, tgt_idx)

  embedded_sources = emb_src.reshape(B, S, D)
  embedded_hypotheses = emb_tgt[:N].reshape(B, S, D)
  embedded_ref0 = emb_tgt[N:2 * N].reshape(B, S, D)
  embedded_ref1 = emb_tgt[2 * N:].reshape(B, S, D)

  masks = jnp.stack([
      sources_attention_mask.reshape(N // 128, 128),
      hypotheses_attention_mask.reshape(N // 128, 128),
      ref0_attention_mask.reshape(N // 128, 128),
      ref1_attention_mask.reshape(N // 128, 128),
  ])
  inv = pl.pallas_call(
      _mask_body,
      out_shape=jax.ShapeDtypeStruct(masks.shape, jnp.bool_),
  )(masks)
  attention_sources = inv[0].reshape(B, S)
  attention_hypotheses = inv[1].reshape(B, S)
  attention_ref0 = inv[2].reshape(B, S)
  attention_ref1 = inv[3].reshape(B, S)

  return (embedded_sources, embedded_hypotheses, embedded_ref0, embedded_ref1,
          attention_sources, attention_hypotheses, attention_ref0,
          attention_ref1)


# speedup vs baseline: 1.1460x; 1.1460x over previous
import functools

import jax
import jax.numpy as jnp
from jax import lax
from jax.experimental import pallas as pl
from jax.experimental.pallas import tpu as pltpu
from jax.experimental.pallas import tpu_sc as plsc

B, S, V, D = 4096, 50, 1000000, 32
N = B * S

_info = plsc.get_sparse_core_info()
NC, NS = _info.num_cores, _info.num_subcores
NW = NC * NS
PER_W = N // NW
CH = 1600
NCH = PER_W // CH


def _sc_gather():
  mesh = plsc.VectorSubcoreMesh(core_axis_name="c", subcore_axis_name="s")

  @functools.partial(
      pl.kernel,
      mesh=mesh,
      compiler_params=pltpu.CompilerParams(use_tc_tiling_on_sc=False),
      out_type=jax.ShapeDtypeStruct((4 * N, D), jnp.float32),
      scratch_types=[
          pltpu.VMEM((CH,), jnp.int32),
          pltpu.VMEM((CH,), jnp.int32),
          pltpu.VMEM((CH, D), jnp.float32),
          pltpu.VMEM((CH, D), jnp.float32),
          pltpu.SemaphoreType.DMA,
          pltpu.SemaphoreType.DMA,
          pltpu.SemaphoreType.DMA,
      ],
  )
  def k(w_src, w_tgt, idx_src, idx_hyp, idx_r0, idx_r1, out,
        idx_v0, idx_v1, rows_v0, rows_v1, gsem, wsem0, wsem1):
    wid = lax.axis_index("s") * NC + lax.axis_index("c")
    idx_v = (idx_v0, idx_v1)
    rows_v = (rows_v0, rows_v1)
    wsem = (wsem0, wsem1)

    segs = ((w_src, idx_src), (w_tgt, idx_hyp), (w_tgt, idx_r0),
            (w_tgt, idx_r1))
    writes = [None, None]
    step = 0
    for seg, (table, idx_hbm) in enumerate(segs):
      for c in range(NCH):
        b = step % 2
        base = wid * PER_W + c * CH
        if writes[b] is not None:
          writes[b].wait()
        pltpu.sync_copy(idx_hbm.at[pl.ds(base, CH)], idx_v[b])
        pltpu.async_copy(table.at[idx_v[b]], rows_v[b], gsem).wait()
        writes[b] = pltpu.async_copy(
            rows_v[b], out.at[pl.ds(seg * N + base, CH)], wsem[b])
        step += 1
    for w in writes:
      if w is not None:
        w.wait()

  return k


def _table_t_body(wt_ref, out_ref):
  out_ref[...] = wt_ref[...].T


_TBLK = 4096


def _transpose_table(wt):
  return pl.pallas_call(
      _table_t_body,
      grid=(pl.cdiv(V, _TBLK),),
      in_specs=[pl.BlockSpec((D, _TBLK), lambda j: (0, j))],
      out_specs=pl.BlockSpec((_TBLK, D), lambda j: (j, 0)),
      out_shape=jax.ShapeDtypeStruct((V, D), jnp.float32),
  )(wt)


def _out_t_body(g_ref, out_ref):
  out_ref[0, 0] = g_ref[0, 0].T


def _transpose_out(g):
  return pl.pallas_call(
      _out_t_body,
      grid=(4, S),
      in_specs=[pl.BlockSpec((1, 1, B, D), lambda i, s: (i, s, 0, 0))],
      out_specs=pl.BlockSpec((1, 1, D, B), lambda i, s: (i, s, 0, 0)),
      out_shape=jax.ShapeDtypeStruct((4, S, D, B), jnp.float32),
  )(g)


def _mask_body(a_ref, b_ref, c_ref, d_ref, oa_ref, ob_ref, oc_ref, od_ref):
  oa_ref[...] = a_ref[...] == 0
  ob_ref[...] = b_ref[...] == 0
  oc_ref[...] = c_ref[...] == 0
  od_ref[...] = d_ref[...] == 0


def kernel(sources_input_ids, sources_attention_mask,
           hypotheses_input_ids, hypotheses_attention_mask,
           ref0_input_ids, ref0_attention_mask,
           ref1_input_ids, ref1_attention_mask,
           W_src, W_tgt):
  idx_src = sources_input_ids.T.reshape(N).astype(jnp.int32)
  idx_hyp = hypotheses_input_ids.T.reshape(N).astype(jnp.int32)
  idx_r0 = ref0_input_ids.T.reshape(N).astype(jnp.int32)
  idx_r1 = ref1_input_ids.T.reshape(N).astype(jnp.int32)

  ws = _transpose_table(W_src.T)
  wt = _transpose_table(W_tgt.T)

  gathered = _sc_gather()(ws, wt, idx_src, idx_hyp, idx_r0, idx_r1)
  outt = _transpose_out(gathered.reshape(4, S, B, D))

  embedded_sources = outt[0].transpose(2, 0, 1)
  embedded_hypotheses = outt[1].transpose(2, 0, 1)
  embedded_ref0 = outt[2].transpose(2, 0, 1)
  embedded_ref1 = outt[3].transpose(2, 0, 1)

  inv = pl.pallas_call(
      _mask_body,
      out_shape=[jax.ShapeDtypeStruct((S, B), jnp.bool_)] * 4,
  )(sources_attention_mask.T, hypotheses_attention_mask.T,
    ref0_attention_mask.T, ref1_attention_mask.T)

  return (embedded_sources, embedded_hypotheses, embedded_ref0, embedded_ref1,
          inv[0].T, inv[1].T, inv[2].T, inv[3].T)


# speedup vs baseline: 1.2757x; 1.1132x over previous
import functools

import jax
import jax.numpy as jnp
from jax import lax
from jax.experimental import pallas as pl
from jax.experimental.pallas import tpu as pltpu
from jax.experimental.pallas import tpu_sc as plsc

B, S, V, D = 4096, 50, 1000000, 32
N = B * S

_info = plsc.get_sparse_core_info()
NC, NS = _info.num_cores, _info.num_subcores
NW = NC * NS
PER_W = N // NW
CH = 1600
NCH = PER_W // CH


def _sc_gather():
  mesh = plsc.VectorSubcoreMesh(core_axis_name="c", subcore_axis_name="s")

  @functools.partial(
      pl.kernel,
      mesh=mesh,
      compiler_params=pltpu.CompilerParams(use_tc_tiling_on_sc=False),
      out_type=jax.ShapeDtypeStruct((4 * N, D), jnp.float32),
      scratch_types=[
          pltpu.VMEM((CH,), jnp.int32),
          pltpu.VMEM((CH,), jnp.int32),
          pltpu.VMEM((CH, D), jnp.float32),
          pltpu.VMEM((CH, D), jnp.float32),
          pltpu.SemaphoreType.DMA,
          pltpu.SemaphoreType.DMA,
          pltpu.SemaphoreType.DMA,
      ],
  )
  def k(w_src, w_tgt, idx_src, idx_hyp, idx_r0, idx_r1, out,
        idx_v0, idx_v1, rows_v0, rows_v1, gsem, wsem0, wsem1):
    wid = lax.axis_index("s") * NC + lax.axis_index("c")
    idx_v = (idx_v0, idx_v1)
    rows_v = (rows_v0, rows_v1)
    wsem = (wsem0, wsem1)

    segs = ((w_src, idx_src), (w_tgt, idx_hyp), (w_tgt, idx_r0),
            (w_tgt, idx_r1))
    writes = [None, None]
    step = 0
    for seg, (table, idx_hbm) in enumerate(segs):
      for c in range(NCH):
        b = step % 2
        base = wid * PER_W + c * CH
        if writes[b] is not None:
          writes[b].wait()
        pltpu.sync_copy(idx_hbm.at[pl.ds(base, CH)], idx_v[b])
        pltpu.async_copy(table.at[idx_v[b]], rows_v[b], gsem).wait()
        writes[b] = pltpu.async_copy(
            rows_v[b], out.at[pl.ds(seg * N + base, CH)], wsem[b])
        step += 1
    for w in writes:
      if w is not None:
        w.wait()

  return k


def _table_t_body(wt_ref, out_ref):
  t = wt_ref[...].T.reshape(_TBLK // 4, 4, D)
  out_ref[...] = jnp.concatenate([t[:, j, :] for j in range(4)], axis=1)


_TBLK = 4096


def _transpose_table(wt):
  return pl.pallas_call(
      _table_t_body,
      grid=(pl.cdiv(V, _TBLK),),
      in_specs=[pl.BlockSpec((D, _TBLK), lambda j: (0, j))],
      out_specs=pl.BlockSpec((_TBLK // 4, 4 * D), lambda j: (j, 0)),
      out_shape=jax.ShapeDtypeStruct((V // 4, 4 * D), jnp.float32),
  )(wt)


def _out_t_body(g_ref, out_ref):
  g = g_ref[0, 0]
  z = jnp.stack([g[:, j * D:(j + 1) * D] for j in range(4)], axis=1)
  out_ref[0, 0] = z.reshape(B, D).T


def _transpose_out(g4):
  return pl.pallas_call(
      _out_t_body,
      grid=(4, S),
      in_specs=[pl.BlockSpec((1, 1, B // 4, 4 * D), lambda i, s: (i, s, 0, 0))],
      out_specs=pl.BlockSpec((1, 1, D, B), lambda i, s: (i, s, 0, 0)),
      out_shape=jax.ShapeDtypeStruct((4, S, D, B), jnp.float32),
  )(g4)


def _mask_body(a_ref, b_ref, c_ref, d_ref, oa_ref, ob_ref, oc_ref, od_ref):
  oa_ref[...] = a_ref[...] == 0
  ob_ref[...] = b_ref[...] == 0
  oc_ref[...] = c_ref[...] == 0
  od_ref[...] = d_ref[...] == 0


def kernel(sources_input_ids, sources_attention_mask,
           hypotheses_input_ids, hypotheses_attention_mask,
           ref0_input_ids, ref0_attention_mask,
           ref1_input_ids, ref1_attention_mask,
           W_src, W_tgt):
  idx_src = sources_input_ids.T.reshape(N).astype(jnp.int32)
  idx_hyp = hypotheses_input_ids.T.reshape(N).astype(jnp.int32)
  idx_r0 = ref0_input_ids.T.reshape(N).astype(jnp.int32)
  idx_r1 = ref1_input_ids.T.reshape(N).astype(jnp.int32)

  ws = _transpose_table(W_src.T).reshape(V, D)
  wt = _transpose_table(W_tgt.T).reshape(V, D)

  gathered = _sc_gather()(ws, wt, idx_src, idx_hyp, idx_r0, idx_r1)
  outt = _transpose_out(gathered.reshape(4, S, B // 4, 4 * D))

  embedded_sources = outt[0].transpose(2, 0, 1)
  embedded_hypotheses = outt[1].transpose(2, 0, 1)
  embedded_ref0 = outt[2].transpose(2, 0, 1)
  embedded_ref1 = outt[3].transpose(2, 0, 1)

  inv = pl.pallas_call(
      _mask_body,
      out_shape=[jax.ShapeDtypeStruct((S, B), jnp.bool_)] * 4,
  )(sources_attention_mask.T, hypotheses_attention_mask.T,
    ref0_attention_mask.T, ref1_attention_mask.T)

  return (embedded_sources, embedded_hypotheses, embedded_ref0, embedded_ref1,
          inv[0].T, inv[1].T, inv[2].T, inv[3].T)


# speedup vs baseline: 2.1673x; 1.6989x over previous
import functools

import jax
import jax.numpy as jnp
from jax import lax
from jax.experimental import pallas as pl
from jax.experimental.pallas import tpu as pltpu
from jax.experimental.pallas import tpu_sc as plsc

B, S, V, D = 4096, 50, 1000000, 32
N = B * S

_info = plsc.get_sparse_core_info()
NC, NS = _info.num_cores, _info.num_subcores
NW = NC * NS
PER_W = N // NW
CH = 1600
NCH = PER_W // CH


def _sc_gather():
  mesh = plsc.VectorSubcoreMesh(core_axis_name="c", subcore_axis_name="s")

  @functools.partial(
      pl.kernel,
      mesh=mesh,
      compiler_params=pltpu.CompilerParams(use_tc_tiling_on_sc=False),
      out_type=jax.ShapeDtypeStruct((4 * N, D), jnp.float32),
      scratch_types=[
          pltpu.VMEM((CH,), jnp.int32),
          pltpu.VMEM((CH,), jnp.int32),
          pltpu.VMEM((CH, D), jnp.float32),
          pltpu.VMEM((CH, D), jnp.float32),
          pltpu.SemaphoreType.DMA,
          pltpu.SemaphoreType.DMA,
          pltpu.SemaphoreType.DMA,
      ],
  )
  def k(w_src, w_tgt, idx_src, idx_hyp, idx_r0, idx_r1, out,
        idx_v0, idx_v1, rows_v0, rows_v1, gsem, wsem0, wsem1):
    wid = lax.axis_index("s") * NC + lax.axis_index("c")
    idx_v = (idx_v0, idx_v1)
    rows_v = (rows_v0, rows_v1)
    wsem = (wsem0, wsem1)

    segs = ((w_src, idx_src), (w_tgt, idx_hyp), (w_tgt, idx_r0),
            (w_tgt, idx_r1))
    writes = [None, None]
    step = 0
    for seg, (table, idx_hbm) in enumerate(segs):
      for c in range(NCH):
        b = step % 2
        base = wid * PER_W + c * CH
        if writes[b] is not None:
          writes[b].wait()
        pltpu.sync_copy(idx_hbm.at[pl.ds(base, CH)], idx_v[b])
        pltpu.async_copy(table.at[idx_v[b]], rows_v[b], gsem).wait()
        writes[b] = pltpu.async_copy(
            rows_v[b], out.at[pl.ds(seg * N + base, CH)], wsem[b])
        step += 1
    for w in writes:
      if w is not None:
        w.wait()

  return k


_TBLK = 8192
_TQ = _TBLK // 4
_TGRID = (V + _TBLK - 1) // _TBLK
_VPAD = _TGRID * _TBLK


def _table_t_body(wt_ref, out_ref):
  parts = [wt_ref[:, c * _TQ:(c + 1) * _TQ].T for c in range(4)]
  out_ref[...] = jnp.concatenate(parts, axis=1)


def _transpose_table(wt):
  return pl.pallas_call(
      _table_t_body,
      grid=(_TGRID,),
      in_specs=[pl.BlockSpec((D, _TBLK), lambda j: (0, j))],
      out_specs=pl.BlockSpec((_TQ, 4 * D), lambda j: (j, 0)),
      out_shape=jax.ShapeDtypeStruct((_VPAD // 4, 4 * D), jnp.float32),
  )(wt)


def _out_t_body(g_ref, out_ref):
  g = g_ref[0, 0]
  parts = [g[:, j * D:(j + 1) * D].T for j in range(4)]
  out_ref[0, 0] = jnp.concatenate(parts, axis=1)


def _transpose_out(g4):
  return pl.pallas_call(
      _out_t_body,
      grid=(4, S),
      in_specs=[pl.BlockSpec((1, 1, B // 4, 4 * D), lambda i, s: (i, s, 0, 0))],
      out_specs=pl.BlockSpec((1, 1, D, B), lambda i, s: (i, s, 0, 0)),
      out_shape=jax.ShapeDtypeStruct((4, S, D, B), jnp.float32),
  )(g4)


def _prep_idx(ids):
  a = ids.T.astype(jnp.int32)
  u = a & (_TBLK - 1)
  a2 = (a - u) + ((u & (_TQ - 1)) << 2) + (u >> 11)
  return a2.reshape(S, 4, B // 4).transpose(0, 2, 1).reshape(N)


def _mask_body(a_ref, b_ref, c_ref, d_ref, oa_ref, ob_ref, oc_ref, od_ref):
  oa_ref[...] = a_ref[...] == 0
  ob_ref[...] = b_ref[...] == 0
  oc_ref[...] = c_ref[...] == 0
  od_ref[...] = d_ref[...] == 0


def kernel(sources_input_ids, sources_attention_mask,
           hypotheses_input_ids, hypotheses_attention_mask,
           ref0_input_ids, ref0_attention_mask,
           ref1_input_ids, ref1_attention_mask,
           W_src, W_tgt):
  idx_src = _prep_idx(sources_input_ids)
  idx_hyp = _prep_idx(hypotheses_input_ids)
  idx_r0 = _prep_idx(ref0_input_ids)
  idx_r1 = _prep_idx(ref1_input_ids)

  ws = _transpose_table(W_src.T).reshape(_VPAD, D)
  wt = _transpose_table(W_tgt.T).reshape(_VPAD, D)

  gathered = _sc_gather()(ws, wt, idx_src, idx_hyp, idx_r0, idx_r1)
  outt = _transpose_out(gathered.reshape(4, S, B // 4, 4 * D))

  embedded_sources = outt[0].transpose(2, 0, 1)
  embedded_hypotheses = outt[1].transpose(2, 0, 1)
  embedded_ref0 = outt[2].transpose(2, 0, 1)
  embedded_ref1 = outt[3].transpose(2, 0, 1)

  inv = pl.pallas_call(
      _mask_body,
      out_shape=[jax.ShapeDtypeStruct((S, B), jnp.bool_)] * 4,
  )(sources_attention_mask.T, hypotheses_attention_mask.T,
    ref0_attention_mask.T, ref1_attention_mask.T)

  return (embedded_sources, embedded_hypotheses, embedded_ref0, embedded_ref1,
          inv[0].T, inv[1].T, inv[2].T, inv[3].T)
